# Initial kernel scaffold; baseline (speedup 1.0000x reference)
import functools

import jax
import jax.numpy as jnp
from jax import lax
from jax.experimental import pallas as pl
from jax.experimental.pallas import tpu as pltpu
from jax.experimental.pallas import tpu_sc as plsc

VOCAB = 1000000
EMB = 32
BATCH = 16384
HIST = 50

B = BATCH * HIST
IDXW = 128
GATHERS = 8
CHUNK = GATHERS * IDXW


def _softplus16(x):
    t = jnp.exp(-jnp.abs(x))
    s = t / (t + 2.0)
    z = s * s
    p = s * (2.0 + z * (2.0 / 3.0 + z * (2.0 / 5.0 + z * (2.0 / 7.0))))
    return jnp.maximum(x, 0.0) + p


def _make_kernel(num_cores, num_subcores):
    nw = num_cores * num_subcores
    assert B % (nw * CHUNK) == 0
    rows_per_w = B // nw
    chunks_per_w = rows_per_w // CHUNK
    idx_rows_per_w = rows_per_w // IDXW

    mesh = plsc.VectorSubcoreMesh(core_axis_name="c", subcore_axis_name="s")

    @functools.partial(
        pl.kernel,
        mesh=mesh,
        out_type=(
            jax.ShapeDtypeStruct((B, EMB), jnp.float32),
            jax.ShapeDtypeStruct((B, EMB), jnp.float32),
        ),
        scratch_types=[
            pltpu.VMEM((GATHERS, IDXW), jnp.int32),
            pltpu.VMEM((CHUNK, EMB), jnp.float32),
            pltpu.VMEM((CHUNK, EMB), jnp.float32),
            pltpu.SemaphoreType.DMA,
            pltpu.SemaphoreType.DMA,
        ],
    )
    def k(ids_hbm, mu_hbm, sig_hbm, mu_out, sig_out,
          idx_v, mu_rows, sig_rows, sem_mu, sem_sig):
        wid = lax.axis_index("s") * num_cores + lax.axis_index("c")
        idx_row0 = wid * idx_rows_per_w
        out_row0 = wid * rows_per_w

        def chunk_body(g, carry):
            pltpu.sync_copy(
                ids_hbm.at[pl.ds(idx_row0 + g * GATHERS, GATHERS), :], idx_v)
            copies = []
            for j in range(GATHERS):
                copies.append(pltpu.async_copy(
                    mu_hbm.at[idx_v.at[j]],
                    mu_rows.at[pl.ds(j * IDXW, IDXW)], sem_mu))
                copies.append(pltpu.async_copy(
                    sig_hbm.at[idx_v.at[j]],
                    sig_rows.at[pl.ds(j * IDXW, IDXW)], sem_sig))
            for c in copies:
                c.wait()

            def row_body(i, carry2):
                for h in range(0, EMB, 16):
                    x = sig_rows[i, pl.ds(h, 16)]
                    sig_rows[i, pl.ds(h, 16)] = _softplus16(x)
                return carry2

            lax.fori_loop(0, CHUNK, row_body, 0, unroll=2)

            base = out_row0 + g * CHUNK
            pltpu.sync_copy(mu_rows, mu_out.at[pl.ds(base, CHUNK)])
            pltpu.sync_copy(sig_rows, sig_out.at[pl.ds(base, CHUNK)])
            return carry

        lax.fori_loop(0, chunks_per_w, chunk_body, 0)

    return k


@jax.jit
def kernel(input_ids, mu_table, sigma_table):
    info = plsc.get_sparse_core_info()
    k = _make_kernel(info.num_cores, info.num_subcores)
    ids2d = input_ids.astype(jnp.int32).reshape(B // IDXW, IDXW)
    mu_flat, sig_flat = k(ids2d, mu_table, sigma_table)
    return (mu_flat.reshape(BATCH, HIST, EMB), sig_flat.reshape(BATCH, HIST, EMB))


# speedup vs baseline: 1.0270x; 1.0270x over previous
import functools

import jax
import jax.numpy as jnp
from jax import lax
from jax.experimental import pallas as pl
from jax.experimental.pallas import tpu as pltpu
from jax.experimental.pallas import tpu_sc as plsc

VOCAB = 1000000
EMB = 32
BATCH = 16384
HIST = 50

B = BATCH * HIST
IDXW = 128
GATHERS = 8
CHUNK = GATHERS * IDXW


def _softplus16(x):
    t = jnp.exp(-jnp.abs(x))
    s = t / (t + 2.0)
    z = s * s
    p = s * (2.0 + z * (2.0 / 3.0 + z * (2.0 / 5.0 + z * (2.0 / 7.0))))
    return jnp.maximum(x, 0.0) + p


def _make_kernel(num_cores, num_subcores):
    nw = num_cores * num_subcores
    assert B % (nw * CHUNK) == 0
    rows_per_w = B // nw
    chunks_per_w = rows_per_w // CHUNK
    idx_rows_per_w = rows_per_w // IDXW

    mesh = plsc.VectorSubcoreMesh(core_axis_name="c", subcore_axis_name="s")

    @functools.partial(
        pl.kernel,
        mesh=mesh,
        compiler_params=pltpu.CompilerParams(use_tc_tiling_on_sc=False),
        out_type=(
            jax.ShapeDtypeStruct((B, EMB), jnp.float32),
            jax.ShapeDtypeStruct((B, EMB), jnp.float32),
        ),
        scratch_types=[
            pltpu.VMEM((GATHERS, IDXW), jnp.int32),
            pltpu.VMEM((CHUNK, EMB), jnp.float32),
            pltpu.VMEM((CHUNK, EMB), jnp.float32),
            pltpu.SemaphoreType.DMA,
            pltpu.SemaphoreType.DMA,
        ],
    )
    def k(ids_hbm, mu_hbm, sig_hbm, mu_out, sig_out,
          idx_v, mu_rows, sig_rows, sem_mu, sem_sig):
        wid = lax.axis_index("s") * num_cores + lax.axis_index("c")
        idx_row0 = wid * idx_rows_per_w
        out_row0 = wid * rows_per_w

        def chunk_body(g, carry):
            pltpu.sync_copy(
                ids_hbm.at[pl.ds(idx_row0 + g * GATHERS, GATHERS), :], idx_v)
            copies = []
            for j in range(GATHERS):
                copies.append(pltpu.async_copy(
                    mu_hbm.at[idx_v.at[j]],
                    mu_rows.at[pl.ds(j * IDXW, IDXW)], sem_mu))
                copies.append(pltpu.async_copy(
                    sig_hbm.at[idx_v.at[j]],
                    sig_rows.at[pl.ds(j * IDXW, IDXW)], sem_sig))
            for c in copies:
                c.wait()

            def row_body(i, carry2):
                for h in range(0, EMB, 16):
                    x = sig_rows[i, pl.ds(h, 16)]
                    sig_rows[i, pl.ds(h, 16)] = _softplus16(x)
                return carry2

            lax.fori_loop(0, CHUNK, row_body, 0, unroll=2)

            base = out_row0 + g * CHUNK
            pltpu.sync_copy(mu_rows, mu_out.at[pl.ds(base, CHUNK)])
            pltpu.sync_copy(sig_rows, sig_out.at[pl.ds(base, CHUNK)])
            return carry

        lax.fori_loop(0, chunks_per_w, chunk_body, 0)

    return k


@jax.jit
def kernel(input_ids, mu_table, sigma_table):
    info = plsc.get_sparse_core_info()
    k = _make_kernel(info.num_cores, info.num_subcores)
    ids2d = input_ids.astype(jnp.int32).reshape(B // IDXW, IDXW)
    mu_flat, sig_flat = k(ids2d, mu_table, sigma_table)
    return (mu_flat.reshape(BATCH, HIST, EMB), sig_flat.reshape(BATCH, HIST, EMB))
